# Initial kernel scaffold; baseline (speedup 1.0000x reference)
import jax
import jax.numpy as jnp
from jax.experimental import pallas as pl


def kernel(x, edge_index, W_enc, b_enc, W1, b1, W2, b2):
    raise NotImplementedError("write your pallas kernel here")


# speedup vs baseline: 23.1545x; 23.1545x over previous
import jax
import jax.numpy as jnp
from jax import lax
from jax.experimental import pallas as pl
from jax.experimental.pallas import tpu as pltpu
from jax.experimental.pallas import tpu_sc as plsc

N = 10000
E = 320000
D = 128
NC = 2
NS = 16
NW = NC * NS
EPT = E // NW
K = 125
NCHUNK = EPT // K
NP = 10240
RPT = NP // NS

RB = 1000
GRID = N // RB

_f32 = jnp.float32
_mesh = plsc.VectorSubcoreMesh(core_axis_name="c", subcore_axis_name="s")


def _deg_body(dst3_hbm, zero_hbm, ones_hbm, out_hbm, dacc, didx_all, ones_v):
    c = lax.axis_index("c")
    s = lax.axis_index("s")
    wid = s * NC + c
    pltpu.sync_copy(dst3_hbm.at[wid], didx_all)
    pltpu.sync_copy(ones_hbm, ones_v)
    pltpu.sync_copy(zero_hbm, dacc.at[pl.ds(s * RPT, RPT)])
    plsc.subcore_barrier()

    @pl.loop(0, NCHUNK)
    def _(g):
        pltpu.sync_copy(ones_v, dacc.at[didx_all.at[g]], add=True)

    plsc.subcore_barrier()
    pltpu.sync_copy(dacc.at[pl.ds(s * RPT, RPT)],
                    out_hbm.at[pl.ds(c * NP + s * RPT, RPT)])


def _edge_body(y_hbm, ei3_hbm, zero_hbm, out_hbm,
               acc, sd0, sd1, rows0, rows1, isem0, isem1, gsem0, gsem1):
    c = lax.axis_index("c")
    s = lax.axis_index("s")
    wid = s * NC + c
    pltpu.sync_copy(zero_hbm, acc.at[pl.ds(s * RPT, RPT)])
    pltpu.sync_copy(ei3_hbm.at[wid, 0], sd0)
    plsc.subcore_barrier()
    pltpu.async_copy(y_hbm.at[sd0.at[0]], rows0, gsem0)
    pltpu.async_copy(ei3_hbm.at[wid, 1], sd1, isem1)

    @pl.loop(0, NCHUNK // 2)
    def _(i):
        g0 = 2 * i
        pltpu.make_async_copy(ei3_hbm.at[wid, g0 + 1], sd1, isem1).wait()
        pltpu.async_copy(y_hbm.at[sd1.at[0]], rows1, gsem1)
        pltpu.make_async_copy(y_hbm.at[sd0.at[0]], rows0, gsem0).wait()
        pltpu.sync_copy(rows0, acc.at[sd0.at[1]], add=True)

        @pl.when(g0 + 2 < NCHUNK)
        def _():
            pltpu.async_copy(ei3_hbm.at[wid, g0 + 2], sd0, isem0)
            pltpu.make_async_copy(ei3_hbm.at[wid, g0 + 2], sd0, isem0).wait()
            pltpu.async_copy(y_hbm.at[sd0.at[0]], rows0, gsem0)

        pltpu.make_async_copy(y_hbm.at[sd1.at[0]], rows1, gsem1).wait()
        pltpu.sync_copy(rows1, acc.at[sd1.at[1]], add=True)

        @pl.when(g0 + 3 < NCHUNK)
        def _():
            pltpu.async_copy(ei3_hbm.at[wid, g0 + 3], sd1, isem1)

    plsc.subcore_barrier()
    pltpu.sync_copy(acc.at[pl.ds(s * RPT, RPT)],
                    out_hbm.at[pl.ds(c * NP + s * RPT, RPT)])


def _make_deg_kernel(interpret=False):
    return pl.kernel(
        _deg_body,
        out_type=jax.ShapeDtypeStruct((NC * NP, D), _f32),
        mesh=_mesh,
        scratch_types=[
            pltpu.VMEM_SHARED((NP, D), _f32),
            pltpu.VMEM((NCHUNK, K), jnp.int32),
            pltpu.VMEM((K, D), _f32),
        ],
        interpret=interpret,
    )


def _make_edge_kernel(interpret=False):
    return pl.kernel(
        _edge_body,
        out_type=jax.ShapeDtypeStruct((NC * NP, D), _f32),
        mesh=_mesh,
        scratch_types=[
            pltpu.VMEM_SHARED((NP, D), _f32),
            pltpu.VMEM((2, K), jnp.int32),
            pltpu.VMEM((2, K), jnp.int32),
            pltpu.VMEM((K, D), _f32),
            pltpu.VMEM((K, D), _f32),
            pltpu.SemaphoreType.DMA,
            pltpu.SemaphoreType.DMA,
            pltpu.SemaphoreType.DMA,
            pltpu.SemaphoreType.DMA,
        ],
        interpret=interpret,
    )


_deg_kernel = _make_deg_kernel()
_edge_kernel = _make_edge_kernel()


def _dot(a, b):
    return jax.lax.dot_general(a, b, (((1,), (0,)), ((), ())),
                               precision=jax.lax.Precision.HIGHEST,
                               preferred_element_type=_f32)


def _enc_body(x_ref, w_ref, b_ref, o_ref):
    o_ref[...] = jnp.maximum(_dot(x_ref[...], w_ref[...]) + b_ref[...], 0.0)


def _dinv_body(d0_ref, d1_ref, o_ref):
    deg = d0_ref[:, 0:1] + d1_ref[:, 0:1] + 1.0
    o_ref[...] = jax.lax.rsqrt(deg)


def _scale_body(h_ref, w_ref, dinv_ref, y_ref):
    y_ref[...] = _dot(h_ref[...], w_ref[...]) * dinv_ref[...]


def _mid_body(a0_ref, a1_ref, y_ref, dinv_ref, b_ref, w_ref, o_ref):
    dinv = dinv_ref[...]
    h = jnp.maximum((a0_ref[...] + a1_ref[...] + y_ref[...]) * dinv
                    + b_ref[...], 0.0)
    o_ref[...] = _dot(h, w_ref[...]) * dinv


def _fin_body(a0_ref, a1_ref, y_ref, dinv_ref, b_ref, o_ref):
    o_ref[...] = jnp.maximum(
        (a0_ref[...] + a1_ref[...] + y_ref[...]) * dinv_ref[...]
        + b_ref[...], 0.0)


def _row_spec(w):
    return pl.BlockSpec((RB, w), lambda i: (i, 0))


def _full_spec(h, w):
    return pl.BlockSpec((h, w), lambda i: (0, 0))


def _enc(x, W, b):
    return pl.pallas_call(
        _enc_body, grid=(GRID,),
        in_specs=[_row_spec(D), _full_spec(D, D), _full_spec(1, D)],
        out_specs=_row_spec(D),
        out_shape=jax.ShapeDtypeStruct((N, D), _f32),
    )(x, W, b)


def _dinv_kernel(d0, d1):
    return pl.pallas_call(
        _dinv_body,
        out_shape=jax.ShapeDtypeStruct((NP, 1), _f32),
    )(d0, d1)


def _scale(h, W, dinv):
    return pl.pallas_call(
        _scale_body, grid=(GRID,),
        in_specs=[_row_spec(D), _full_spec(D, D), _row_spec(1)],
        out_specs=_row_spec(D),
        out_shape=jax.ShapeDtypeStruct((N, D), _f32),
    )(h, W, dinv)


def _mid(a0, a1, y, dinv, b, W):
    return pl.pallas_call(
        _mid_body, grid=(GRID,),
        in_specs=[_row_spec(D), _row_spec(D), _row_spec(D), _row_spec(1),
                  _full_spec(1, D), _full_spec(D, D)],
        out_specs=_row_spec(D),
        out_shape=jax.ShapeDtypeStruct((N, D), _f32),
    )(a0, a1, y, dinv, b, W)


def _fin(a0, a1, y, dinv, b):
    return pl.pallas_call(
        _fin_body, grid=(GRID,),
        in_specs=[_row_spec(D), _row_spec(D), _row_spec(D), _row_spec(1),
                  _full_spec(1, D)],
        out_specs=_row_spec(D),
        out_shape=jax.ShapeDtypeStruct((N, D), _f32),
    )(a0, a1, y, dinv, b)


def kernel(x, edge_index, W_enc, b_enc, W1, b1, W2, b2):
    src = edge_index[0].astype(jnp.int32)
    dst = edge_index[1].astype(jnp.int32)
    src3 = src.reshape(NW, NCHUNK, K)
    dst3 = dst.reshape(NW, NCHUNK, K)
    ei3 = jnp.stack([src3, dst3], axis=2)
    zD = jnp.zeros((RPT, D), _f32)
    oD = jnp.ones((K, D), _f32)

    deg_parts = _deg_kernel(dst3, zD, oD)
    dinv_full = _dinv_kernel(deg_parts[:NP], deg_parts[NP:])
    dinv = dinv_full[:N]

    h = _enc(x, W_enc, b_enc.reshape(1, D))
    y1 = _scale(h, W1, dinv)

    e1 = _edge_kernel(y1, ei3, zD)
    y2 = _mid(e1[:N], e1[NP:NP + N], y1, dinv, b1.reshape(1, D), W2)

    e2 = _edge_kernel(y2, ei3, zD)
    out = _fin(e2[:N], e2[NP:NP + N], y2, dinv, b2.reshape(1, D))
    return out


# speedup vs baseline: 25.0323x; 1.0811x over previous
import jax
import jax.numpy as jnp
from jax import lax
from jax.experimental import pallas as pl
from jax.experimental.pallas import tpu as pltpu
from jax.experimental.pallas import tpu_sc as plsc

N = 10000
E = 320000
D = 128
NC = 2
NS = 16
NW = NC * NS
EPT = E // NW
K = 125
NCHUNK = EPT // K
NP = 10240
RPT = NP // NS

RB = 1000
GRID = N // RB

_f32 = jnp.float32
_mesh = plsc.VectorSubcoreMesh(core_axis_name="c", subcore_axis_name="s")


def _deg_body(dst3_hbm, zero_hbm, ones_hbm, out_hbm, dacc, didx_all, ones_v):
    c = lax.axis_index("c")
    s = lax.axis_index("s")
    wid = s * NC + c
    pltpu.sync_copy(dst3_hbm.at[wid], didx_all)
    pltpu.sync_copy(ones_hbm, ones_v)
    pltpu.sync_copy(zero_hbm, dacc.at[pl.ds(s * RPT, RPT)])
    plsc.subcore_barrier()

    @pl.loop(0, NCHUNK)
    def _(g):
        pltpu.sync_copy(ones_v, dacc.at[didx_all.at[g]], add=True)

    plsc.subcore_barrier()
    pltpu.sync_copy(dacc.at[pl.ds(s * RPT, RPT)],
                    out_hbm.at[c, pl.ds(s * RPT, RPT)])


def _edge_body(y_hbm, ei3_hbm, zero_hbm, out_hbm,
               acc, sd0, sd1, rows0, rows1, isem0, isem1, gsem0, gsem1):
    c = lax.axis_index("c")
    s = lax.axis_index("s")
    wid = s * NC + c
    pltpu.sync_copy(zero_hbm, acc.at[pl.ds(s * RPT, RPT)])
    pltpu.sync_copy(ei3_hbm.at[wid, 0], sd0)
    plsc.subcore_barrier()
    pltpu.async_copy(y_hbm.at[sd0.at[0]], rows0, gsem0)
    pltpu.async_copy(ei3_hbm.at[wid, 1], sd1, isem1)

    @pl.loop(0, NCHUNK // 2)
    def _(i):
        g0 = 2 * i
        pltpu.make_async_copy(ei3_hbm.at[wid, g0 + 1], sd1, isem1).wait()
        pltpu.async_copy(y_hbm.at[sd1.at[0]], rows1, gsem1)
        pltpu.make_async_copy(y_hbm.at[sd0.at[0]], rows0, gsem0).wait()
        pltpu.sync_copy(rows0, acc.at[sd0.at[1]], add=True)

        @pl.when(g0 + 2 < NCHUNK)
        def _():
            pltpu.async_copy(ei3_hbm.at[wid, g0 + 2], sd0, isem0)
            pltpu.make_async_copy(ei3_hbm.at[wid, g0 + 2], sd0, isem0).wait()
            pltpu.async_copy(y_hbm.at[sd0.at[0]], rows0, gsem0)

        pltpu.make_async_copy(y_hbm.at[sd1.at[0]], rows1, gsem1).wait()
        pltpu.sync_copy(rows1, acc.at[sd1.at[1]], add=True)

        @pl.when(g0 + 3 < NCHUNK)
        def _():
            pltpu.async_copy(ei3_hbm.at[wid, g0 + 3], sd1, isem1)

    plsc.subcore_barrier()
    pltpu.sync_copy(acc.at[pl.ds(s * RPT, RPT)],
                    out_hbm.at[c, pl.ds(s * RPT, RPT)])


def _make_deg_kernel(interpret=False):
    return pl.kernel(
        _deg_body,
        out_type=jax.ShapeDtypeStruct((NC, NP, D), _f32),
        mesh=_mesh,
        scratch_types=[
            pltpu.VMEM_SHARED((NP, D), _f32),
            pltpu.VMEM((NCHUNK, K), jnp.int32),
            pltpu.VMEM((K, D), _f32),
        ],
        interpret=interpret,
    )


def _make_edge_kernel(interpret=False):
    return pl.kernel(
        _edge_body,
        out_type=jax.ShapeDtypeStruct((NC, NP, D), _f32),
        mesh=_mesh,
        scratch_types=[
            pltpu.VMEM_SHARED((NP, D), _f32),
            pltpu.VMEM((2, K), jnp.int32),
            pltpu.VMEM((2, K), jnp.int32),
            pltpu.VMEM((K, D), _f32),
            pltpu.VMEM((K, D), _f32),
            pltpu.SemaphoreType.DMA,
            pltpu.SemaphoreType.DMA,
            pltpu.SemaphoreType.DMA,
            pltpu.SemaphoreType.DMA,
        ],
        interpret=interpret,
    )


_deg_kernel = _make_deg_kernel()
_edge_kernel = _make_edge_kernel()


def _dot(a, b):
    return jax.lax.dot_general(a, b, (((1,), (0,)), ((), ())),
                               precision=jax.lax.Precision.HIGHEST,
                               preferred_element_type=_f32)


def _enc_body(x_ref, w_ref, b_ref, o_ref):
    o_ref[...] = jnp.maximum(_dot(x_ref[...], w_ref[...]) + b_ref[...], 0.0)


def _scale_body(h_ref, w_ref, d0_ref, d1_ref, y_ref, dinv_ref):
    deg = d0_ref[0, :, 0:1] + d1_ref[0, :, 0:1] + 1.0
    dinv = jax.lax.rsqrt(deg)
    dinv_ref[...] = dinv
    y_ref[...] = _dot(h_ref[...], w_ref[...]) * dinv


def _mid_body(a0_ref, a1_ref, y_ref, dinv_ref, b_ref, w_ref, o_ref):
    dinv = dinv_ref[...]
    h = jnp.maximum((a0_ref[0] + a1_ref[0] + y_ref[...]) * dinv
                    + b_ref[...], 0.0)
    o_ref[...] = _dot(h, w_ref[...]) * dinv


def _fin_body(a0_ref, a1_ref, y_ref, dinv_ref, b_ref, o_ref):
    o_ref[...] = jnp.maximum(
        (a0_ref[0] + a1_ref[0] + y_ref[...]) * dinv_ref[...]
        + b_ref[...], 0.0)


def _row_spec(w):
    return pl.BlockSpec((RB, w), lambda i: (i, 0))


def _part_spec(core):
    return pl.BlockSpec((1, RB, D), lambda i: (core, i, 0))


def _full_spec(h, w):
    return pl.BlockSpec((h, w), lambda i: (0, 0))


def _enc(x, W, b):
    return pl.pallas_call(
        _enc_body, grid=(GRID,),
        in_specs=[_row_spec(D), _full_spec(D, D), _full_spec(1, D)],
        out_specs=_row_spec(D),
        out_shape=jax.ShapeDtypeStruct((N, D), _f32),
    )(x, W, b)


def _scale(h, W, parts):
    return pl.pallas_call(
        _scale_body, grid=(GRID,),
        in_specs=[_row_spec(D), _full_spec(D, D), _part_spec(0), _part_spec(1)],
        out_specs=(_row_spec(D), _row_spec(1)),
        out_shape=(jax.ShapeDtypeStruct((N, D), _f32),
                   jax.ShapeDtypeStruct((N, 1), _f32)),
    )(h, W, parts, parts)


def _mid(e, y, dinv, b, W):
    return pl.pallas_call(
        _mid_body, grid=(GRID,),
        in_specs=[_part_spec(0), _part_spec(1), _row_spec(D), _row_spec(1),
                  _full_spec(1, D), _full_spec(D, D)],
        out_specs=_row_spec(D),
        out_shape=jax.ShapeDtypeStruct((N, D), _f32),
    )(e, e, y, dinv, b, W)


def _fin(e, y, dinv, b):
    return pl.pallas_call(
        _fin_body, grid=(GRID,),
        in_specs=[_part_spec(0), _part_spec(1), _row_spec(D), _row_spec(1),
                  _full_spec(1, D)],
        out_specs=_row_spec(D),
        out_shape=jax.ShapeDtypeStruct((N, D), _f32),
    )(e, e, y, dinv, b)


def kernel(x, edge_index, W_enc, b_enc, W1, b1, W2, b2):
    src = edge_index[0].astype(jnp.int32)
    dst = edge_index[1].astype(jnp.int32)
    src3 = src.reshape(NW, NCHUNK, K)
    dst3 = dst.reshape(NW, NCHUNK, K)
    ei3 = jnp.stack([src3, dst3], axis=2)
    zD = jnp.zeros((RPT, D), _f32)
    oD = jnp.ones((K, D), _f32)

    deg_parts = _deg_kernel(dst3, zD, oD)
    h = _enc(x, W_enc, b_enc.reshape(1, D))
    y1, dinv = _scale(h, W1, deg_parts)

    e1 = _edge_kernel(y1, ei3, zD)
    y2 = _mid(e1, y1, dinv, b1.reshape(1, D), W2)

    e2 = _edge_kernel(y2, ei3, zD)
    out = _fin(e2, y2, dinv, b2.reshape(1, D))
    return out


# speedup vs baseline: 27.9504x; 1.1166x over previous
import dataclasses

import jax
import jax.numpy as jnp
from jax import lax
from jax.experimental import pallas as pl
from jax.experimental.pallas import tpu as pltpu
from jax.experimental.pallas import tpu_sc as plsc

N = 10000
E = 320000
D = 128
NC = 2
NS = 16
NW = NC * NS
EPT = E // NW
K = 125
NCHUNK = EPT // K
NP = 10240
RPT = NP // NS

RB = 1000
GRID = N // RB

_f32 = jnp.float32
_mesh = plsc.VectorSubcoreMesh(core_axis_name="c", subcore_axis_name="s")

DEGW = 16
_notile = dataclasses.replace(pltpu.CompilerParams(),
                              use_tc_tiling_on_sc=False)


def _deg_body(dst3_hbm, zero_hbm, ones_hbm, out_hbm, dacc, didx_all, ones_v):
    c = lax.axis_index("c")
    s = lax.axis_index("s")
    wid = s * NC + c
    pltpu.sync_copy(dst3_hbm.at[wid], didx_all)
    pltpu.sync_copy(ones_hbm, ones_v)
    pltpu.sync_copy(zero_hbm, dacc.at[pl.ds(s * RPT, RPT)])
    plsc.subcore_barrier()

    @pl.loop(0, NCHUNK)
    def _(g):
        pltpu.sync_copy(ones_v, dacc.at[didx_all.at[g]], add=True)

    plsc.subcore_barrier()
    pltpu.sync_copy(dacc.at[pl.ds(s * RPT, RPT)],
                    out_hbm.at[c, pl.ds(s * RPT, RPT)])


def _edge_body(y_hbm, ei3_hbm, zero_hbm, out_hbm,
               acc, sd0, sd1, rows0, rows1, isem0, isem1, gsem0, gsem1):
    c = lax.axis_index("c")
    s = lax.axis_index("s")
    wid = s * NC + c
    pltpu.sync_copy(zero_hbm, acc.at[pl.ds(s * RPT, RPT)])
    pltpu.sync_copy(ei3_hbm.at[wid, 0], sd0)
    plsc.subcore_barrier()
    pltpu.async_copy(y_hbm.at[sd0.at[0]], rows0, gsem0)
    pltpu.async_copy(ei3_hbm.at[wid, 1], sd1, isem1)

    @pl.loop(0, NCHUNK // 2)
    def _(i):
        g0 = 2 * i
        pltpu.make_async_copy(ei3_hbm.at[wid, g0 + 1], sd1, isem1).wait()
        pltpu.async_copy(y_hbm.at[sd1.at[0]], rows1, gsem1)
        pltpu.make_async_copy(y_hbm.at[sd0.at[0]], rows0, gsem0).wait()
        pltpu.sync_copy(rows0, acc.at[sd0.at[1]], add=True)

        @pl.when(g0 + 2 < NCHUNK)
        def _():
            pltpu.async_copy(ei3_hbm.at[wid, g0 + 2], sd0, isem0)
            pltpu.make_async_copy(ei3_hbm.at[wid, g0 + 2], sd0, isem0).wait()
            pltpu.async_copy(y_hbm.at[sd0.at[0]], rows0, gsem0)

        pltpu.make_async_copy(y_hbm.at[sd1.at[0]], rows1, gsem1).wait()
        pltpu.sync_copy(rows1, acc.at[sd1.at[1]], add=True)

        @pl.when(g0 + 3 < NCHUNK)
        def _():
            pltpu.async_copy(ei3_hbm.at[wid, g0 + 3], sd1, isem1)

    plsc.subcore_barrier()
    pltpu.sync_copy(acc.at[pl.ds(s * RPT, RPT)],
                    out_hbm.at[c, pl.ds(s * RPT, RPT)])


def _make_deg_kernel(interpret=False):
    return pl.kernel(
        _deg_body,
        out_type=jax.ShapeDtypeStruct((NC, NP, DEGW), _f32),
        mesh=_mesh,
        scratch_types=[
            pltpu.VMEM_SHARED((NP, DEGW), _f32),
            pltpu.VMEM((NCHUNK, K), jnp.int32),
            pltpu.VMEM((K, DEGW), _f32),
        ],
        compiler_params=_notile,
        interpret=interpret,
    )


def _make_edge_kernel(interpret=False):
    return pl.kernel(
        _edge_body,
        out_type=jax.ShapeDtypeStruct((NC, NP, D), _f32),
        mesh=_mesh,
        scratch_types=[
            pltpu.VMEM_SHARED((NP, D), _f32),
            pltpu.VMEM((2, K), jnp.int32),
            pltpu.VMEM((2, K), jnp.int32),
            pltpu.VMEM((K, D), _f32),
            pltpu.VMEM((K, D), _f32),
            pltpu.SemaphoreType.DMA,
            pltpu.SemaphoreType.DMA,
            pltpu.SemaphoreType.DMA,
            pltpu.SemaphoreType.DMA,
        ],
        interpret=interpret,
    )


_deg_kernel = _make_deg_kernel()
_edge_kernel = _make_edge_kernel()


def _dot(a, b):
    return jax.lax.dot_general(a, b, (((1,), (0,)), ((), ())),
                               precision=jax.lax.Precision.HIGHEST,
                               preferred_element_type=_f32)


def _enc_body(x_ref, w_ref, b_ref, o_ref):
    o_ref[...] = jnp.maximum(_dot(x_ref[...], w_ref[...]) + b_ref[...], 0.0)


def _scale_body(h_ref, w_ref, d0_ref, d1_ref, y_ref, dinv_ref):
    deg = d0_ref[0, :, 0:1] + d1_ref[0, :, 0:1] + 1.0
    dinv = jax.lax.rsqrt(deg)
    dinv_ref[...] = dinv
    y_ref[...] = _dot(h_ref[...], w_ref[...]) * dinv


def _mid_body(a0_ref, a1_ref, y_ref, dinv_ref, b_ref, w_ref, o_ref):
    dinv = dinv_ref[...]
    h = jnp.maximum((a0_ref[0] + a1_ref[0] + y_ref[...]) * dinv
                    + b_ref[...], 0.0)
    o_ref[...] = _dot(h, w_ref[...]) * dinv


def _fin_body(a0_ref, a1_ref, y_ref, dinv_ref, b_ref, o_ref):
    o_ref[...] = jnp.maximum(
        (a0_ref[0] + a1_ref[0] + y_ref[...]) * dinv_ref[...]
        + b_ref[...], 0.0)


def _row_spec(w):
    return pl.BlockSpec((RB, w), lambda i: (i, 0))


def _part_spec(core, w=D):
    return pl.BlockSpec((1, RB, w), lambda i: (core, i, 0))


def _full_spec(h, w):
    return pl.BlockSpec((h, w), lambda i: (0, 0))


def _enc(x, W, b):
    return pl.pallas_call(
        _enc_body, grid=(GRID,),
        in_specs=[_row_spec(D), _full_spec(D, D), _full_spec(1, D)],
        out_specs=_row_spec(D),
        out_shape=jax.ShapeDtypeStruct((N, D), _f32),
    )(x, W, b)


def _scale(h, W, parts):
    return pl.pallas_call(
        _scale_body, grid=(GRID,),
        in_specs=[_row_spec(D), _full_spec(D, D),
                  _part_spec(0, DEGW), _part_spec(1, DEGW)],
        out_specs=(_row_spec(D), _row_spec(1)),
        out_shape=(jax.ShapeDtypeStruct((N, D), _f32),
                   jax.ShapeDtypeStruct((N, 1), _f32)),
    )(h, W, parts, parts)


def _mid(e, y, dinv, b, W):
    return pl.pallas_call(
        _mid_body, grid=(GRID,),
        in_specs=[_part_spec(0), _part_spec(1), _row_spec(D), _row_spec(1),
                  _full_spec(1, D), _full_spec(D, D)],
        out_specs=_row_spec(D),
        out_shape=jax.ShapeDtypeStruct((N, D), _f32),
    )(e, e, y, dinv, b, W)


def _fin(e, y, dinv, b):
    return pl.pallas_call(
        _fin_body, grid=(GRID,),
        in_specs=[_part_spec(0), _part_spec(1), _row_spec(D), _row_spec(1),
                  _full_spec(1, D)],
        out_specs=_row_spec(D),
        out_shape=jax.ShapeDtypeStruct((N, D), _f32),
    )(e, e, y, dinv, b)


def kernel(x, edge_index, W_enc, b_enc, W1, b1, W2, b2):
    src = edge_index[0].astype(jnp.int32)
    dst = edge_index[1].astype(jnp.int32)
    src3 = src.reshape(NW, NCHUNK, K)
    dst3 = dst.reshape(NW, NCHUNK, K)
    ei3 = jnp.stack([src3, dst3], axis=2)
    zD = jnp.zeros((RPT, D), _f32)
    z16 = jnp.zeros((RPT, DEGW), _f32)
    o16 = jnp.ones((K, DEGW), _f32)

    deg_parts = _deg_kernel(dst3, z16, o16)
    h = _enc(x, W_enc, b_enc.reshape(1, D))
    y1, dinv = _scale(h, W1, deg_parts)

    e1 = _edge_kernel(y1, ei3, zD)
    y2 = _mid(e1, y1, dinv, b1.reshape(1, D), W2)

    e2 = _edge_kernel(y2, ei3, zD)
    out = _fin(e2, y2, dinv, b2.reshape(1, D))
    return out


# speedup vs baseline: 28.9671x; 1.0364x over previous
import dataclasses

import jax
import jax.numpy as jnp
from jax import lax
from jax.experimental import pallas as pl
from jax.experimental.pallas import tpu as pltpu
from jax.experimental.pallas import tpu_sc as plsc

N = 10000
E = 320000
D = 128
NC = 2
NS = 16
NW = NC * NS
EPT = E // NW
K = 125
NCHUNK = EPT // K
NP = 10240
RPT = NP // NS

RB = 1000
GRID = N // RB

_f32 = jnp.float32
_mesh = plsc.VectorSubcoreMesh(core_axis_name="c", subcore_axis_name="s")

DEGW = 16
_notile = dataclasses.replace(pltpu.CompilerParams(),
                              use_tc_tiling_on_sc=False)


def _deg_body(dst3_hbm, zero_hbm, ones_hbm, out_hbm, dacc, didx_all, ones_v):
    c = lax.axis_index("c")
    s = lax.axis_index("s")
    wid = s * NC + c
    pltpu.sync_copy(dst3_hbm.at[wid], didx_all)
    pltpu.sync_copy(ones_hbm, ones_v)
    pltpu.sync_copy(zero_hbm, dacc.at[pl.ds(s * RPT, RPT)])
    plsc.subcore_barrier()

    @pl.loop(0, NCHUNK)
    def _(g):
        pltpu.sync_copy(ones_v, dacc.at[didx_all.at[g]], add=True)

    plsc.subcore_barrier()
    pltpu.sync_copy(dacc.at[pl.ds(s * RPT, RPT)],
                    out_hbm.at[c, pl.ds(s * RPT, RPT)])


def _edge_body(y_hbm, ei3_hbm, zero_hbm, out_hbm,
               acc, sd0, sd1, rows0, rows1, isem0, isem1, gsem0, gsem1):
    c = lax.axis_index("c")
    s = lax.axis_index("s")
    wid = s * NC + c
    pltpu.sync_copy(zero_hbm, acc.at[pl.ds(s * RPT, RPT)])
    pltpu.sync_copy(ei3_hbm.at[wid, 0], sd0)
    plsc.subcore_barrier()
    pltpu.async_copy(y_hbm.at[sd0.at[0]], rows0, gsem0)
    pltpu.async_copy(ei3_hbm.at[wid, 1], sd1, isem1)

    @pl.loop(0, NCHUNK // 2)
    def _(i):
        g0 = 2 * i
        pltpu.make_async_copy(ei3_hbm.at[wid, g0 + 1], sd1, isem1).wait()
        pltpu.async_copy(y_hbm.at[sd1.at[0]], rows1, gsem1)
        pltpu.make_async_copy(y_hbm.at[sd0.at[0]], rows0, gsem0).wait()
        pltpu.sync_copy(rows0, acc.at[sd0.at[1]], add=True)

        @pl.when(g0 + 2 < NCHUNK)
        def _():
            pltpu.async_copy(ei3_hbm.at[wid, g0 + 2], sd0, isem0)
            pltpu.make_async_copy(ei3_hbm.at[wid, g0 + 2], sd0, isem0).wait()
            pltpu.async_copy(y_hbm.at[sd0.at[0]], rows0, gsem0)

        pltpu.make_async_copy(y_hbm.at[sd1.at[0]], rows1, gsem1).wait()
        pltpu.sync_copy(rows1, acc.at[sd1.at[1]], add=True)

        @pl.when(g0 + 3 < NCHUNK)
        def _():
            pltpu.async_copy(ei3_hbm.at[wid, g0 + 3], sd1, isem1)

    plsc.subcore_barrier()
    pltpu.sync_copy(acc.at[pl.ds(s * RPT, RPT)],
                    out_hbm.at[c, pl.ds(s * RPT, RPT)])


def _make_deg_kernel(interpret=False):
    return pl.kernel(
        _deg_body,
        out_type=jax.ShapeDtypeStruct((NC, NP, DEGW), _f32),
        mesh=_mesh,
        scratch_types=[
            pltpu.VMEM_SHARED((NP, DEGW), _f32),
            pltpu.VMEM((NCHUNK, K), jnp.int32),
            pltpu.VMEM((K, DEGW), _f32),
        ],
        compiler_params=_notile,
        interpret=interpret,
    )


def _make_edge_kernel(interpret=False):
    return pl.kernel(
        _edge_body,
        out_type=jax.ShapeDtypeStruct((NC, NP, D), _f32),
        mesh=_mesh,
        scratch_types=[
            pltpu.VMEM_SHARED((NP, D), _f32),
            pltpu.VMEM((2, K), jnp.int32),
            pltpu.VMEM((2, K), jnp.int32),
            pltpu.VMEM((K, D), _f32),
            pltpu.VMEM((K, D), _f32),
            pltpu.SemaphoreType.DMA,
            pltpu.SemaphoreType.DMA,
            pltpu.SemaphoreType.DMA,
            pltpu.SemaphoreType.DMA,
        ],
        interpret=interpret,
    )


_deg_kernel = _make_deg_kernel()
_edge_kernel = _make_edge_kernel()


def _dot(a, b):
    return jax.lax.dot_general(a, b, (((1,), (0,)), ((), ())),
                               preferred_element_type=_f32)


def _scale_body(x_ref, we_ref, be_ref, w_ref, d0_ref, d1_ref, y_ref, dinv_ref):
    deg = d0_ref[0, :, 0:1] + d1_ref[0, :, 0:1] + 1.0
    dinv = jax.lax.rsqrt(deg)
    dinv_ref[...] = dinv
    h = jnp.maximum(_dot(x_ref[...], we_ref[...]) + be_ref[...], 0.0)
    y_ref[...] = _dot(h, w_ref[...]) * dinv


def _mid_body(a0_ref, a1_ref, y_ref, dinv_ref, b_ref, w_ref, o_ref):
    dinv = dinv_ref[...]
    h = jnp.maximum((a0_ref[0] + a1_ref[0] + y_ref[...]) * dinv
                    + b_ref[...], 0.0)
    o_ref[...] = _dot(h, w_ref[...]) * dinv


def _fin_body(a0_ref, a1_ref, y_ref, dinv_ref, b_ref, o_ref):
    o_ref[...] = jnp.maximum(
        (a0_ref[0] + a1_ref[0] + y_ref[...]) * dinv_ref[...]
        + b_ref[...], 0.0)


def _row_spec(w):
    return pl.BlockSpec((RB, w), lambda i: (i, 0))


def _part_spec(core, w=D):
    return pl.BlockSpec((1, RB, w), lambda i: (core, i, 0))


def _full_spec(h, w):
    return pl.BlockSpec((h, w), lambda i: (0, 0))


def _scale(x, We, be, W, parts):
    return pl.pallas_call(
        _scale_body, grid=(GRID,),
        in_specs=[_row_spec(D), _full_spec(D, D), _full_spec(1, D),
                  _full_spec(D, D), _part_spec(0, DEGW), _part_spec(1, DEGW)],
        out_specs=(_row_spec(D), _row_spec(1)),
        out_shape=(jax.ShapeDtypeStruct((N, D), _f32),
                   jax.ShapeDtypeStruct((N, 1), _f32)),
    )(x, We, be, W, parts, parts)


def _mid(e, y, dinv, b, W):
    return pl.pallas_call(
        _mid_body, grid=(GRID,),
        in_specs=[_part_spec(0), _part_spec(1), _row_spec(D), _row_spec(1),
                  _full_spec(1, D), _full_spec(D, D)],
        out_specs=_row_spec(D),
        out_shape=jax.ShapeDtypeStruct((N, D), _f32),
    )(e, e, y, dinv, b, W)


def _fin(e, y, dinv, b):
    return pl.pallas_call(
        _fin_body, grid=(GRID,),
        in_specs=[_part_spec(0), _part_spec(1), _row_spec(D), _row_spec(1),
                  _full_spec(1, D)],
        out_specs=_row_spec(D),
        out_shape=jax.ShapeDtypeStruct((N, D), _f32),
    )(e, e, y, dinv, b)


def kernel(x, edge_index, W_enc, b_enc, W1, b1, W2, b2):
    src = edge_index[0].astype(jnp.int32)
    dst = edge_index[1].astype(jnp.int32)
    src3 = src.reshape(NW, NCHUNK, K)
    dst3 = dst.reshape(NW, NCHUNK, K)
    ei3 = jnp.stack([src3, dst3], axis=2)
    zD = jnp.zeros((RPT, D), _f32)
    z16 = jnp.zeros((RPT, DEGW), _f32)
    o16 = jnp.ones((K, DEGW), _f32)

    deg_parts = _deg_kernel(dst3, z16, o16)
    y1, dinv = _scale(x, W_enc, b_enc.reshape(1, D), W1, deg_parts)

    e1 = _edge_kernel(y1, ei3, zD)
    y2 = _mid(e1, y1, dinv, b1.reshape(1, D), W2)

    e2 = _edge_kernel(y2, ei3, zD)
    out = _fin(e2, y2, dinv, b2.reshape(1, D))
    return out


# speedup vs baseline: 31.9923x; 1.1044x over previous
import dataclasses

import jax
import jax.numpy as jnp
from jax import lax
from jax.experimental import pallas as pl
from jax.experimental.pallas import tpu as pltpu
from jax.experimental.pallas import tpu_sc as plsc

N = 10000
E = 320000
D = 128
NC = 2
NS = 16
NW = NC * NS
EPT = E // NW
K = 125
NCHUNK = EPT // K
NP = 10240
RPT = NP // NS

RB = 1000
GRID = N // RB

_f32 = jnp.float32
_mesh = plsc.VectorSubcoreMesh(core_axis_name="c", subcore_axis_name="s")

DEGW = 16
_notile = dataclasses.replace(pltpu.CompilerParams(),
                              use_tc_tiling_on_sc=False)


def _deg_body(dst3_hbm, zero_hbm, ones_hbm, out_hbm, dacc, didx_all, ones_v):
    c = lax.axis_index("c")
    s = lax.axis_index("s")
    wid = s * NC + c
    pltpu.sync_copy(dst3_hbm.at[wid], didx_all)
    pltpu.sync_copy(ones_hbm, ones_v)
    pltpu.sync_copy(zero_hbm, dacc.at[pl.ds(s * RPT, RPT)])
    plsc.subcore_barrier()

    @pl.loop(0, NCHUNK)
    def _(g):
        pltpu.sync_copy(ones_v, dacc.at[didx_all.at[g]], add=True)

    plsc.subcore_barrier()
    pltpu.sync_copy(dacc.at[pl.ds(s * RPT, RPT)],
                    out_hbm.at[c, pl.ds(s * RPT, RPT)])


def _edge_body(y_hbm, ei3_hbm, zero_hbm, out_hbm,
               acc, sd0, sd1, sd2, sd3, rows0, rows1,
               isem0, isem1, isem2, isem3, gsem0, gsem1):
    c = lax.axis_index("c")
    s = lax.axis_index("s")
    wid = s * NC + c
    pltpu.sync_copy(zero_hbm, acc.at[pl.ds(s * RPT, RPT)])
    pltpu.sync_copy(ei3_hbm.at[wid, 0], sd0)
    plsc.subcore_barrier()
    pltpu.async_copy(y_hbm.at[sd0.at[0]], rows0, gsem0)
    pltpu.async_copy(ei3_hbm.at[wid, 1], sd1, isem1)
    pltpu.async_copy(ei3_hbm.at[wid, 2], sd2, isem2)
    pltpu.async_copy(ei3_hbm.at[wid, 3], sd3, isem3)

    @pl.loop(0, NCHUNK // 4)
    def _(j):
        b = 4 * j

        pltpu.make_async_copy(ei3_hbm.at[wid, b + 1], sd1, isem1).wait()
        pltpu.async_copy(y_hbm.at[sd1.at[0]], rows1, gsem1)

        pltpu.make_async_copy(y_hbm.at[sd0.at[0]], rows0, gsem0).wait()
        pltpu.sync_copy(rows0, acc.at[sd0.at[1]], add=True)

        @pl.when(b + 4 < NCHUNK)
        def _():
            pltpu.async_copy(ei3_hbm.at[wid, b + 4], sd0, isem0)

        pltpu.make_async_copy(ei3_hbm.at[wid, b + 2], sd2, isem2).wait()
        pltpu.async_copy(y_hbm.at[sd2.at[0]], rows0, gsem0)

        pltpu.make_async_copy(y_hbm.at[sd1.at[0]], rows1, gsem1).wait()
        pltpu.sync_copy(rows1, acc.at[sd1.at[1]], add=True)

        @pl.when(b + 5 < NCHUNK)
        def _():
            pltpu.async_copy(ei3_hbm.at[wid, b + 5], sd1, isem1)

        pltpu.make_async_copy(ei3_hbm.at[wid, b + 3], sd3, isem3).wait()
        pltpu.async_copy(y_hbm.at[sd3.at[0]], rows1, gsem1)

        pltpu.make_async_copy(y_hbm.at[sd2.at[0]], rows0, gsem0).wait()
        pltpu.sync_copy(rows0, acc.at[sd2.at[1]], add=True)

        @pl.when(b + 6 < NCHUNK)
        def _():
            pltpu.async_copy(ei3_hbm.at[wid, b + 6], sd2, isem2)

        @pl.when(b + 4 < NCHUNK)
        def _():
            pltpu.make_async_copy(ei3_hbm.at[wid, b + 4], sd0, isem0).wait()
            pltpu.async_copy(y_hbm.at[sd0.at[0]], rows0, gsem0)

        pltpu.make_async_copy(y_hbm.at[sd3.at[0]], rows1, gsem1).wait()
        pltpu.sync_copy(rows1, acc.at[sd3.at[1]], add=True)

        @pl.when(b + 7 < NCHUNK)
        def _():
            pltpu.async_copy(ei3_hbm.at[wid, b + 7], sd3, isem3)

    plsc.subcore_barrier()
    pltpu.sync_copy(acc.at[pl.ds(s * RPT, RPT)],
                    out_hbm.at[c, pl.ds(s * RPT, RPT)])


def _make_deg_kernel(interpret=False):
    return pl.kernel(
        _deg_body,
        out_type=jax.ShapeDtypeStruct((NC, NP, DEGW), _f32),
        mesh=_mesh,
        scratch_types=[
            pltpu.VMEM_SHARED((NP, DEGW), _f32),
            pltpu.VMEM((NCHUNK, K), jnp.int32),
            pltpu.VMEM((K, DEGW), _f32),
        ],
        compiler_params=_notile,
        interpret=interpret,
    )


def _make_edge_kernel(interpret=False):
    return pl.kernel(
        _edge_body,
        out_type=jax.ShapeDtypeStruct((NC, NP, D), _f32),
        mesh=_mesh,
        scratch_types=[
            pltpu.VMEM_SHARED((NP, D), _f32),
            pltpu.VMEM((2, K), jnp.int32),
            pltpu.VMEM((2, K), jnp.int32),
            pltpu.VMEM((2, K), jnp.int32),
            pltpu.VMEM((2, K), jnp.int32),
            pltpu.VMEM((K, D), _f32),
            pltpu.VMEM((K, D), _f32),
            pltpu.SemaphoreType.DMA,
            pltpu.SemaphoreType.DMA,
            pltpu.SemaphoreType.DMA,
            pltpu.SemaphoreType.DMA,
            pltpu.SemaphoreType.DMA,
            pltpu.SemaphoreType.DMA,
        ],
        interpret=interpret,
    )


_deg_kernel = _make_deg_kernel()
_edge_kernel = _make_edge_kernel()


def _dot(a, b):
    return jax.lax.dot_general(a, b, (((1,), (0,)), ((), ())),
                               preferred_element_type=_f32)


def _scale_body(x_ref, we_ref, be_ref, w_ref, d0_ref, d1_ref, y_ref, dinv_ref):
    deg = d0_ref[0, :, 0:1] + d1_ref[0, :, 0:1] + 1.0
    dinv = jax.lax.rsqrt(deg)
    dinv_ref[...] = dinv
    h = jnp.maximum(_dot(x_ref[...], we_ref[...]) + be_ref[...], 0.0)
    y_ref[...] = _dot(h, w_ref[...]) * dinv


def _mid_body(a0_ref, a1_ref, y_ref, dinv_ref, b_ref, w_ref, o_ref):
    dinv = dinv_ref[...]
    h = jnp.maximum((a0_ref[0] + a1_ref[0] + y_ref[...]) * dinv
                    + b_ref[...], 0.0)
    o_ref[...] = _dot(h, w_ref[...]) * dinv


def _fin_body(a0_ref, a1_ref, y_ref, dinv_ref, b_ref, o_ref):
    o_ref[...] = jnp.maximum(
        (a0_ref[0] + a1_ref[0] + y_ref[...]) * dinv_ref[...]
        + b_ref[...], 0.0)


def _row_spec(w):
    return pl.BlockSpec((RB, w), lambda i: (i, 0))


def _part_spec(core, w=D):
    return pl.BlockSpec((1, RB, w), lambda i: (core, i, 0))


def _full_spec(h, w):
    return pl.BlockSpec((h, w), lambda i: (0, 0))


def _scale(x, We, be, W, parts):
    return pl.pallas_call(
        _scale_body, grid=(GRID,),
        in_specs=[_row_spec(D), _full_spec(D, D), _full_spec(1, D),
                  _full_spec(D, D), _part_spec(0, DEGW), _part_spec(1, DEGW)],
        out_specs=(_row_spec(D), _row_spec(1)),
        out_shape=(jax.ShapeDtypeStruct((N, D), _f32),
                   jax.ShapeDtypeStruct((N, 1), _f32)),
    )(x, We, be, W, parts, parts)


def _mid(e, y, dinv, b, W):
    return pl.pallas_call(
        _mid_body, grid=(GRID,),
        in_specs=[_part_spec(0), _part_spec(1), _row_spec(D), _row_spec(1),
                  _full_spec(1, D), _full_spec(D, D)],
        out_specs=_row_spec(D),
        out_shape=jax.ShapeDtypeStruct((N, D), _f32),
    )(e, e, y, dinv, b, W)


def _fin(e, y, dinv, b):
    return pl.pallas_call(
        _fin_body, grid=(GRID,),
        in_specs=[_part_spec(0), _part_spec(1), _row_spec(D), _row_spec(1),
                  _full_spec(1, D)],
        out_specs=_row_spec(D),
        out_shape=jax.ShapeDtypeStruct((N, D), _f32),
    )(e, e, y, dinv, b)


def kernel(x, edge_index, W_enc, b_enc, W1, b1, W2, b2):
    src = edge_index[0].astype(jnp.int32)
    dst = edge_index[1].astype(jnp.int32)
    src3 = src.reshape(NW, NCHUNK, K)
    dst3 = dst.reshape(NW, NCHUNK, K)
    ei3 = jnp.stack([src3, dst3], axis=2)
    zD = jnp.zeros((RPT, D), _f32)
    z16 = jnp.zeros((RPT, DEGW), _f32)
    o16 = jnp.ones((K, DEGW), _f32)

    deg_parts = _deg_kernel(dst3, z16, o16)
    y1, dinv = _scale(x, W_enc, b_enc.reshape(1, D), W1, deg_parts)

    e1 = _edge_kernel(y1, ei3, zD)
    y2 = _mid(e1, y1, dinv, b1.reshape(1, D), W2)

    e2 = _edge_kernel(y2, ei3, zD)
    out = _fin(e2, y2, dinv, b2.reshape(1, D))
    return out


# speedup vs baseline: 32.4140x; 1.0132x over previous
import dataclasses

import jax
import jax.numpy as jnp
from jax import lax
from jax.experimental import pallas as pl
from jax.experimental.pallas import tpu as pltpu
from jax.experimental.pallas import tpu_sc as plsc

N = 10000
E = 320000
D = 128
NC = 2
NS = 16
NW = NC * NS
EPT = E // NW
K = 125
NCHUNK = EPT // K
NP = 10240
RPT = NP // NS

RB = 1000
GRID = N // RB

_f32 = jnp.float32
_mesh = plsc.VectorSubcoreMesh(core_axis_name="c", subcore_axis_name="s")

DEGW = 16
_notile = dataclasses.replace(pltpu.CompilerParams(),
                              use_tc_tiling_on_sc=False)


def _deg_body(dst3_hbm, zero_hbm, ones_hbm, out_hbm, dacc, didx_all, ones_v,
              ssem):
    c = lax.axis_index("c")
    s = lax.axis_index("s")
    wid = s * NC + c
    pltpu.sync_copy(dst3_hbm.at[wid], didx_all)
    pltpu.sync_copy(ones_hbm, ones_v)
    pltpu.sync_copy(zero_hbm, dacc.at[pl.ds(s * RPT, RPT)])
    plsc.subcore_barrier()

    @pl.loop(0, NCHUNK)
    def _(g):
        pltpu.async_copy(ones_v, dacc.at[didx_all.at[g]], ssem, add=True)

        @pl.when(g >= 4)
        def _():
            pltpu.make_async_copy(ones_v, dacc.at[didx_all.at[g - 4]],
                                  ssem).wait()

    @pl.loop(NCHUNK - 4, NCHUNK)
    def _(g):
        pltpu.make_async_copy(ones_v, dacc.at[didx_all.at[g]], ssem).wait()

    plsc.subcore_barrier()
    pltpu.sync_copy(dacc.at[pl.ds(s * RPT, RPT)],
                    out_hbm.at[c, pl.ds(s * RPT, RPT)])


def _edge_body(y_hbm, ei3_hbm, zero_hbm, out_hbm,
               acc, sd0, sd1, sd2, sd3, rows0, rows1,
               isem0, isem1, isem2, isem3, gsem0, gsem1):
    c = lax.axis_index("c")
    s = lax.axis_index("s")
    wid = s * NC + c
    pltpu.sync_copy(zero_hbm, acc.at[pl.ds(s * RPT, RPT)])
    pltpu.sync_copy(ei3_hbm.at[wid, 0], sd0)
    plsc.subcore_barrier()
    pltpu.async_copy(y_hbm.at[sd0.at[0]], rows0, gsem0)
    pltpu.async_copy(ei3_hbm.at[wid, 1], sd1, isem1)
    pltpu.async_copy(ei3_hbm.at[wid, 2], sd2, isem2)
    pltpu.async_copy(ei3_hbm.at[wid, 3], sd3, isem3)

    @pl.loop(0, NCHUNK // 4)
    def _(j):
        b = 4 * j

        pltpu.make_async_copy(ei3_hbm.at[wid, b + 1], sd1, isem1).wait()
        pltpu.async_copy(y_hbm.at[sd1.at[0]], rows1, gsem1)

        pltpu.make_async_copy(y_hbm.at[sd0.at[0]], rows0, gsem0).wait()
        pltpu.sync_copy(rows0, acc.at[sd0.at[1]], add=True)

        @pl.when(b + 4 < NCHUNK)
        def _():
            pltpu.async_copy(ei3_hbm.at[wid, b + 4], sd0, isem0)

        pltpu.make_async_copy(ei3_hbm.at[wid, b + 2], sd2, isem2).wait()
        pltpu.async_copy(y_hbm.at[sd2.at[0]], rows0, gsem0)

        pltpu.make_async_copy(y_hbm.at[sd1.at[0]], rows1, gsem1).wait()
        pltpu.sync_copy(rows1, acc.at[sd1.at[1]], add=True)

        @pl.when(b + 5 < NCHUNK)
        def _():
            pltpu.async_copy(ei3_hbm.at[wid, b + 5], sd1, isem1)

        pltpu.make_async_copy(ei3_hbm.at[wid, b + 3], sd3, isem3).wait()
        pltpu.async_copy(y_hbm.at[sd3.at[0]], rows1, gsem1)

        pltpu.make_async_copy(y_hbm.at[sd2.at[0]], rows0, gsem0).wait()
        pltpu.sync_copy(rows0, acc.at[sd2.at[1]], add=True)

        @pl.when(b + 6 < NCHUNK)
        def _():
            pltpu.async_copy(ei3_hbm.at[wid, b + 6], sd2, isem2)

        @pl.when(b + 4 < NCHUNK)
        def _():
            pltpu.make_async_copy(ei3_hbm.at[wid, b + 4], sd0, isem0).wait()
            pltpu.async_copy(y_hbm.at[sd0.at[0]], rows0, gsem0)

        pltpu.make_async_copy(y_hbm.at[sd3.at[0]], rows1, gsem1).wait()
        pltpu.sync_copy(rows1, acc.at[sd3.at[1]], add=True)

        @pl.when(b + 7 < NCHUNK)
        def _():
            pltpu.async_copy(ei3_hbm.at[wid, b + 7], sd3, isem3)

    plsc.subcore_barrier()
    pltpu.sync_copy(acc.at[pl.ds(s * RPT, RPT)],
                    out_hbm.at[c, pl.ds(s * RPT, RPT)])


def _make_deg_kernel(interpret=False):
    return pl.kernel(
        _deg_body,
        out_type=jax.ShapeDtypeStruct((NC, NP, DEGW), _f32),
        mesh=_mesh,
        scratch_types=[
            pltpu.VMEM_SHARED((NP, DEGW), _f32),
            pltpu.VMEM((NCHUNK, K), jnp.int32),
            pltpu.VMEM((K, DEGW), _f32),
            pltpu.SemaphoreType.DMA,
        ],
        compiler_params=_notile,
        interpret=interpret,
    )


def _make_edge_kernel(interpret=False):
    return pl.kernel(
        _edge_body,
        out_type=jax.ShapeDtypeStruct((NC, NP, D), _f32),
        mesh=_mesh,
        scratch_types=[
            pltpu.VMEM_SHARED((NP, D), _f32),
            pltpu.VMEM((2, K), jnp.int32),
            pltpu.VMEM((2, K), jnp.int32),
            pltpu.VMEM((2, K), jnp.int32),
            pltpu.VMEM((2, K), jnp.int32),
            pltpu.VMEM((K, D), _f32),
            pltpu.VMEM((K, D), _f32),
            pltpu.SemaphoreType.DMA,
            pltpu.SemaphoreType.DMA,
            pltpu.SemaphoreType.DMA,
            pltpu.SemaphoreType.DMA,
            pltpu.SemaphoreType.DMA,
            pltpu.SemaphoreType.DMA,
        ],
        interpret=interpret,
    )


_deg_kernel = _make_deg_kernel()
_edge_kernel = _make_edge_kernel()


def _dot(a, b):
    return jax.lax.dot_general(a, b, (((1,), (0,)), ((), ())),
                               preferred_element_type=_f32)


def _scale_body(x_ref, we_ref, be_ref, w_ref, d0_ref, d1_ref, y_ref, dinv_ref):
    deg = d0_ref[0, :, 0:1] + d1_ref[0, :, 0:1] + 1.0
    dinv = jax.lax.rsqrt(deg)
    dinv_ref[...] = dinv
    h = jnp.maximum(_dot(x_ref[...], we_ref[...]) + be_ref[...], 0.0)
    y_ref[...] = _dot(h, w_ref[...]) * dinv


def _mid_body(a0_ref, a1_ref, y_ref, dinv_ref, b_ref, w_ref, o_ref):
    dinv = dinv_ref[...]
    h = jnp.maximum((a0_ref[0] + a1_ref[0] + y_ref[...]) * dinv
                    + b_ref[...], 0.0)
    o_ref[...] = _dot(h, w_ref[...]) * dinv


def _fin_body(a0_ref, a1_ref, y_ref, dinv_ref, b_ref, o_ref):
    o_ref[...] = jnp.maximum(
        (a0_ref[0] + a1_ref[0] + y_ref[...]) * dinv_ref[...]
        + b_ref[...], 0.0)


def _row_spec(w):
    return pl.BlockSpec((RB, w), lambda i: (i, 0))


def _part_spec(core, w=D):
    return pl.BlockSpec((1, RB, w), lambda i: (core, i, 0))


def _full_spec(h, w):
    return pl.BlockSpec((h, w), lambda i: (0, 0))


def _scale(x, We, be, W, parts):
    return pl.pallas_call(
        _scale_body, grid=(GRID,),
        in_specs=[_row_spec(D), _full_spec(D, D), _full_spec(1, D),
                  _full_spec(D, D), _part_spec(0, DEGW), _part_spec(1, DEGW)],
        out_specs=(_row_spec(D), _row_spec(1)),
        out_shape=(jax.ShapeDtypeStruct((N, D), _f32),
                   jax.ShapeDtypeStruct((N, 1), _f32)),
    )(x, We, be, W, parts, parts)


def _mid(e, y, dinv, b, W):
    return pl.pallas_call(
        _mid_body, grid=(GRID,),
        in_specs=[_part_spec(0), _part_spec(1), _row_spec(D), _row_spec(1),
                  _full_spec(1, D), _full_spec(D, D)],
        out_specs=_row_spec(D),
        out_shape=jax.ShapeDtypeStruct((N, D), _f32),
    )(e, e, y, dinv, b, W)


def _fin(e, y, dinv, b):
    return pl.pallas_call(
        _fin_body, grid=(GRID,),
        in_specs=[_part_spec(0), _part_spec(1), _row_spec(D), _row_spec(1),
                  _full_spec(1, D)],
        out_specs=_row_spec(D),
        out_shape=jax.ShapeDtypeStruct((N, D), _f32),
    )(e, e, y, dinv, b)


def kernel(x, edge_index, W_enc, b_enc, W1, b1, W2, b2):
    src = edge_index[0].astype(jnp.int32)
    dst = edge_index[1].astype(jnp.int32)
    src3 = src.reshape(NW, NCHUNK, K)
    dst3 = dst.reshape(NW, NCHUNK, K)
    ei3 = jnp.stack([src3, dst3], axis=2)
    zD = jnp.zeros((RPT, D), _f32)
    z16 = jnp.zeros((RPT, DEGW), _f32)
    o16 = jnp.ones((K, DEGW), _f32)

    deg_parts = _deg_kernel(dst3, z16, o16)
    y1, dinv = _scale(x, W_enc, b_enc.reshape(1, D), W1, deg_parts)

    e1 = _edge_kernel(y1, ei3, zD)
    y2 = _mid(e1, y1, dinv, b1.reshape(1, D), W2)

    e2 = _edge_kernel(y2, ei3, zD)
    out = _fin(e2, y2, dinv, b2.reshape(1, D))
    return out
